# baseline (speedup 1.0000x reference)
import jax
import jax.numpy as jnp
from jax import lax
from jax.experimental import pallas as pl
from jax.experimental.pallas import tpu as pltpu, tpu_sc as plsc

_FIELD = 3847
_NF = 26
_K = 16
_VOCAB = _FIELD * _NF
_B = 1024

_VC = 1024
_NVC = (_VOCAB + _VC - 1) // _VC
_VP = _NVC * _VC
_TW = 256

_INFO = plsc.get_sparse_core_info()
_NC, _NS = _INFO.num_cores, _INFO.num_subcores
_NW = _NC * _NS
_SPW = _B // _NW
_C = 4
_NCHUNK = _SPW // _C
_GPC = _C * _NF


def _to_u32(xbf16):
    return lax.convert_element_type(
        lax.bitcast_convert_type(xbf16, jnp.uint16), jnp.uint32)


def _pack_body(embT_ref, w_ref, out_ref):
    half = _NF // 2 * _K
    blk4 = embT_ref[...].reshape(_NF // 2, 2, _K, _VC)
    ev = blk4[:, 0].reshape(half, _VC).T.astype(jnp.bfloat16)
    od = blk4[:, 1].reshape(half, _VC).T.astype(jnp.bfloat16)
    w = jnp.broadcast_to(w_ref[0][:, None], (_VC, _K)).astype(jnp.bfloat16)
    zt = jnp.zeros((_VC, _TW - half - _K), jnp.bfloat16)
    ev_all = jnp.concatenate([ev, w, zt], axis=1)
    od_all = jnp.concatenate(
        [od, jnp.zeros((_VC, _TW - half), jnp.bfloat16)], axis=1)
    word = _to_u32(ev_all) | (_to_u32(od_all) << 16)
    out_ref[...] = lax.bitcast_convert_type(word, jnp.int32)


def _pack_table(embT, W):
    return pl.pallas_call(
        _pack_body,
        grid=(_NVC,),
        in_specs=[
            pl.BlockSpec((_NF, _K, _VC), lambda j: (0, 0, j)),
            pl.BlockSpec((1, _VC), lambda j: (0, j)),
        ],
        out_specs=pl.BlockSpec((_VC, _TW), lambda j: (j, 0)),
        out_shape=jax.ShapeDtypeStruct((_VP, _TW), jnp.int32),
    )(embT, W)


_GDN = lax.GatherDimensionNumbers(
    offset_dims=(), collapsed_slice_dims=(0,), start_index_map=(0,))


def _lane_permute(v, xor_mask):
    perm = (jnp.arange(16, dtype=jnp.int32) ^ xor_mask)[:, None]
    return lax.gather(v, perm, _GDN, (1,),
                      mode=lax.GatherScatterMode.PROMISE_IN_BOUNDS)


def _sc_body(idx_hbm, tbl_hbm, out_hbm,
             idx_a, rows_a, idx_b, rows_b, out_v, sem_a, sem_b):
    wid = lax.axis_index("s") * _NC + lax.axis_index("c")

    def fire(k, idx_v, rows_v, sem):
        base_s = wid * _SPW + k * _C
        pltpu.sync_copy(idx_hbm.at[pl.ds(base_s * _NF, _GPC)], idx_v)
        pltpu.async_copy(tbl_hbm.at[idx_v], rows_v, sem)

    def drain(rows_v, sem):
        pltpu.make_async_copy(tbl_hbm.at[pl.ds(0, _GPC)], rows_v, sem).wait()

    def compute(k, rows_v):
        def up(row, fp):
            x = rows_v[row, pl.ds(_K * fp, _K)]
            a = lax.bitcast_convert_type(lax.shift_left(x, 16), jnp.float32)
            b = lax.bitcast_convert_type(
                lax.bitwise_and(x, jnp.int32(-65536)), jnp.float32)
            return a, b

        def samp_body(ci, carry2):
            r0 = ci * _NF
            accs = [jnp.zeros((16,), jnp.float32) for _ in range(4)]
            t = 0

            def acc(v):
                nonlocal t
                accs[t & 3] = accs[t & 3] + v
                t += 1

            for fp in range(_NF // 2):
                a0, a1 = up(r0 + 2 * fp + 1, fp)
                b0, b1 = up(r0 + 2 * fp, fp)
                acc(a0 * b1)
                for gp in range(fp + 1, _NF // 2):
                    ua0, ua1 = up(r0 + 2 * gp, fp)
                    ub0, ub1 = up(r0 + 2 * gp + 1, fp)
                    uc0, uc1 = up(r0 + 2 * fp, gp)
                    ud0, ud1 = up(r0 + 2 * fp + 1, gp)
                    acc(ua0 * uc0)
                    acc(ub0 * uc1)
                    acc(ua1 * ud0)
                    acc(ub1 * ud1)
            wacc = up(r0, _NF // 2)[0]
            for j in range(1, _NF):
                wacc = wacc + up(r0 + j, _NF // 2)[0]
            tot = (accs[0] + accs[1]) + (accs[2] + accs[3]) + wacc * (1.0 / 16.0)
            for step in (8, 4, 2, 1):
                tot = tot + _lane_permute(tot, step)
            out_v[k * _C + ci, :] = tot
            return carry2

        lax.fori_loop(0, _C, samp_body, 0)

    fire(0, idx_a, rows_a, sem_a)

    def pair_body(k2, carry):
        k = 2 * k2
        fire(k + 1, idx_b, rows_b, sem_b)
        drain(rows_a, sem_a)
        compute(k, rows_a)

        @pl.when(k2 < _NCHUNK // 2 - 1)
        def _():
            fire(k + 2, idx_a, rows_a, sem_a)

        drain(rows_b, sem_b)
        compute(k + 1, rows_b)
        return carry

    lax.fori_loop(0, _NCHUNK // 2, pair_body, 0)
    pltpu.sync_copy(out_v, out_hbm.at[pl.ds(wid * _SPW, _SPW)])


@jax.jit
def kernel(x, emb, W, b):
    x = x.astype(jnp.int32)
    offs = (jnp.arange(_NF, dtype=jnp.int32) * _FIELD)[None, :]
    xo = x + offs
    idx = xo.reshape(_B * _NF)
    tbl = _pack_table(jnp.transpose(emb, (0, 2, 1)), W)

    mesh = plsc.VectorSubcoreMesh(core_axis_name="c", subcore_axis_name="s")
    run = pl.kernel(
        _sc_body, mesh=mesh,
        out_type=jax.ShapeDtypeStruct((_B, _K), jnp.float32),
        scratch_types=[
            pltpu.VMEM((_GPC,), jnp.int32),
            pltpu.VMEM((_GPC, _TW), jnp.int32),
            pltpu.VMEM((_GPC,), jnp.int32),
            pltpu.VMEM((_GPC, _TW), jnp.int32),
            pltpu.VMEM((_SPW, _K), jnp.float32),
            pltpu.SemaphoreType.DMA,
            pltpu.SemaphoreType.DMA,
        ],
    )
    out16 = run(idx, tbl)
    return out16[:, 0] + b[0]
